# baseline (speedup 1.0000x reference)
import functools

import jax
import jax.numpy as jnp
from jax.experimental import pallas as pl
from jax.experimental.pallas import tpu as pltpu

_VMEM_LIMIT = 60 * 1024 * 1024
_ROW_PAD = 256


def _round_up(x, m):
    return ((x + m - 1) // m) * m


def _pad_rows(x, n_pad):
    n = x.shape[0]
    if n == n_pad:
        return x
    return jnp.pad(x, ((0, n_pad - n),) + ((0, 0),) * (x.ndim - 1))


def _gcn_kernel(bm, n_strips, adj_ref, entx_ref, w1_ref, b1_ref, w2_ref,
                b2_ref, gph_o, adj_c_ref, hw2_ref):
    i = pl.program_id(0)
    row0 = pl.multiple_of(i * bm, bm)

    @pl.when(i < n_strips)
    def _():
        a16 = adj_ref[...].astype(jnp.bfloat16)
        adj_c_ref[pl.ds(row0, bm), :] = a16
        xw1 = jnp.dot(entx_ref[...], w1_ref[...],
                      preferred_element_type=jnp.float32)
        acc = jnp.dot(a16, xw1.astype(jnp.bfloat16),
                      preferred_element_type=jnp.float32)
        h = jnp.maximum(acc + b1_ref[...], 0.0)
        hw2_ref[pl.ds(row0, bm), :] = jnp.dot(
            h, w2_ref[...], preferred_element_type=jnp.float32
        ).astype(jnp.bfloat16)

    @pl.when(i == n_strips)
    def _():
        hw2 = hw2_ref[...]
        for j in range(n_strips):
            a16 = adj_c_ref[j * bm:(j + 1) * bm, :]
            gph_o[j * bm:(j + 1) * bm, :] = jnp.dot(
                a16, hw2, preferred_element_type=jnp.float32) + b2_ref[...]


def _modal_kernel(wn_ref, gph_x, img_x, rel_x, att_x, name_x, char_x, ocr_x,
                  img_w, img_b, rel_w, rel_b, att_w, att_b,
                  name_w, name_b, char_w, char_b, ocr_w, ocr_b,
                  img_o, rel_o, att_o, name_o, char_o, joint_o, ocr_o):
    def proj(x_ref, w_ref, b_ref):
        return jnp.dot(x_ref[...], w_ref[...],
                       preferred_element_type=jnp.float32) + b_ref[...]

    def l2n(e):
        ss = jnp.sum(e * e, axis=1, keepdims=True)
        return e * jax.lax.rsqrt(jnp.maximum(ss, 1e-24))

    img_e = proj(img_x, img_w, img_b)
    rel_e = proj(rel_x, rel_w, rel_b)
    att_e = proj(att_x, att_w, att_b)
    name_e = proj(name_x, name_w, name_b)
    char_e = proj(char_x, char_w, char_b)
    ocr_e = proj(ocr_x, ocr_w, ocr_b)

    img_o[...] = img_e
    rel_o[...] = rel_e
    att_o[...] = att_e
    name_o[...] = name_e
    char_o[...] = char_e
    ocr_o[...] = ocr_e

    off = 0
    for e, wn_idx in ((img_e, 0), (att_e, 1), (rel_e, 2), (gph_x[...], 3),
                      (name_e, 4), (char_e, 5), (ocr_e, 6)):
        d = e.shape[1]
        joint_o[:, off:off + d] = l2n(e) * wn_ref[wn_idx]
        off += d


def kernel(entity_emb, gc1_w, gc1_b, gc2_w, gc2_b, rel_w, rel_b, att_w, att_b,
           img_w, img_b, name_w, name_b, char_w, char_b, ocr_w, ocr_b,
           fusion_w, input_idx, adj, img_features, rel_features, att_features,
           name_features, char_features, ocr_features):
    n = adj.shape[0]
    n_pad = _round_up(max(n, _ROW_PAD), _ROW_PAD)
    bm = 512 if n_pad % 512 == 0 else 256

    ent_x = _pad_rows(entity_emb[input_idx], n_pad)
    adj_p = _pad_rows(
        jnp.pad(adj, ((0, 0), (0, n_pad - n))) if n != n_pad else adj, n_pad)
    img_x = _pad_rows(img_features, n_pad)
    rel_x = _pad_rows(rel_features, n_pad)
    att_x = _pad_rows(att_features, n_pad)
    name_x = _pad_rows(name_features, n_pad)
    char_x = _pad_rows(char_features, n_pad)
    ocr_x = _pad_rows(ocr_features, n_pad)

    d_in = ent_x.shape[1]
    nhid = gc1_w.shape[1]
    nout = gc2_w.shape[1]
    b1 = gc1_b.reshape(1, -1)
    b2 = gc2_b.reshape(1, -1)

    n_strips = n_pad // bm
    gph = pl.pallas_call(
        functools.partial(_gcn_kernel, bm, n_strips),
        grid=(n_strips + 1,),
        in_specs=[
            pl.BlockSpec((bm, n_pad),
                         lambda i: (jnp.minimum(i, n_strips - 1), 0)),
            pl.BlockSpec((n_pad, d_in), lambda i: (0, 0)),
            pl.BlockSpec((d_in, nhid), lambda i: (0, 0)),
            pl.BlockSpec((1, nhid), lambda i: (0, 0)),
            pl.BlockSpec((nhid, nout), lambda i: (0, 0)),
            pl.BlockSpec((1, nout), lambda i: (0, 0)),
        ],
        out_specs=pl.BlockSpec((n_pad, nout), lambda i: (0, 0)),
        out_shape=jax.ShapeDtypeStruct((n_pad, nout), jnp.float32),
        scratch_shapes=[pltpu.VMEM((n_pad, n_pad), jnp.bfloat16),
                        pltpu.VMEM((n_pad, nout), jnp.bfloat16)],
        compiler_params=pltpu.CompilerParams(
            dimension_semantics=("arbitrary",),
            vmem_limit_bytes=_VMEM_LIMIT),
    )(adj_p, ent_x, gc1_w, b1, gc2_w, b2)

    weight_norm = jax.nn.softmax(fusion_w, axis=0)[:, 0]

    d_img = img_w.shape[1]
    d_rel = rel_w.shape[1]
    d_att = att_w.shape[1]
    d_name = name_w.shape[1]
    d_char = char_w.shape[1]
    d_ocr = ocr_w.shape[1]
    d_joint = d_img + d_att + d_rel + nout + d_name + d_char + d_ocr

    bmm = 1024 if n_pad % 1024 == 0 else bm

    def row_spec(d):
        return pl.BlockSpec((bmm, d), lambda i: (i, 0))

    def pinned(shape):
        return pl.BlockSpec(shape, lambda i: (0, 0))

    in_specs = [
        pl.BlockSpec(memory_space=pltpu.MemorySpace.SMEM),
        row_spec(nout),
        row_spec(img_x.shape[1]), row_spec(rel_x.shape[1]),
        row_spec(att_x.shape[1]), row_spec(name_x.shape[1]),
        row_spec(char_x.shape[1]), row_spec(ocr_x.shape[1]),
        pinned(img_w.shape), pinned((1, d_img)),
        pinned(rel_w.shape), pinned((1, d_rel)),
        pinned(att_w.shape), pinned((1, d_att)),
        pinned(name_w.shape), pinned((1, d_name)),
        pinned(char_w.shape), pinned((1, d_char)),
        pinned(ocr_w.shape), pinned((1, d_ocr)),
    ]
    out_specs = (row_spec(d_img), row_spec(d_rel), row_spec(d_att),
                 row_spec(d_name), row_spec(d_char), row_spec(d_joint),
                 row_spec(d_ocr))
    out_shape = tuple(jax.ShapeDtypeStruct((n_pad, d), jnp.float32)
                      for d in (d_img, d_rel, d_att, d_name, d_char,
                                d_joint, d_ocr))

    outs = pl.pallas_call(
        _modal_kernel,
        grid=(n_pad // bmm,),
        in_specs=in_specs,
        out_specs=out_specs,
        out_shape=out_shape,
        compiler_params=pltpu.CompilerParams(
            dimension_semantics=("arbitrary",),
            vmem_limit_bytes=_VMEM_LIMIT),
    )(weight_norm, gph,
      img_x, rel_x, att_x, name_x, char_x, ocr_x,
      img_w, img_b.reshape(1, -1), rel_w, rel_b.reshape(1, -1),
      att_w, att_b.reshape(1, -1), name_w, name_b.reshape(1, -1),
      char_w, char_b.reshape(1, -1), ocr_w, ocr_b.reshape(1, -1))

    img_o, rel_o, att_o, name_o, char_o, joint_o, ocr_o = outs
    return (gph[:n], img_o[:n], rel_o[:n], att_o[:n], name_o[:n],
            char_o[:n], joint_o[:n], ocr_o[:n])


# speedup vs baseline: 1.0125x; 1.0125x over previous
import functools

import jax
import jax.numpy as jnp
from jax.experimental import pallas as pl
from jax.experimental.pallas import tpu as pltpu

_VMEM_LIMIT = 60 * 1024 * 1024
_ROW_PAD = 256


def _round_up(x, m):
    return ((x + m - 1) // m) * m


def _pad_rows(x, n_pad):
    n = x.shape[0]
    if n == n_pad:
        return x
    return jnp.pad(x, ((0, n_pad - n),) + ((0, 0),) * (x.ndim - 1))


def _gcn_kernel(bm, n_strips, adj_ref, entx_ref, w1_ref, b1_ref, w2_ref,
                b2_ref, gph_o, adj_c_ref, hw2_ref):
    i = pl.program_id(0)
    row0 = pl.multiple_of(i * bm, bm)

    @pl.when(i < n_strips)
    def _():
        a16 = adj_ref[...].astype(jnp.bfloat16)
        adj_c_ref[pl.ds(row0, bm), :] = a16
        xw1 = jnp.dot(entx_ref[...], w1_ref[...],
                      preferred_element_type=jnp.float32)
        acc = jnp.dot(a16, xw1.astype(jnp.bfloat16),
                      preferred_element_type=jnp.float32)
        h = jnp.maximum(acc + b1_ref[...], 0.0)
        hw2_ref[pl.ds(row0, bm), :] = jnp.dot(
            h, w2_ref[...], preferred_element_type=jnp.float32
        ).astype(jnp.bfloat16)

    @pl.when(i == n_strips)
    def _():
        hw2 = hw2_ref[...]
        for j in range(n_strips):
            a16 = adj_c_ref[j * bm:(j + 1) * bm, :]
            gph_o[j * bm:(j + 1) * bm, :] = jnp.dot(
                a16, hw2, preferred_element_type=jnp.float32) + b2_ref[...]


def _modal_kernel(wn_ref, gph_x, img_x, rel_x, att_x, name_x, char_x, ocr_x,
                  img_w, img_b, rel_w, rel_b, att_w, att_b,
                  name_w, name_b, char_w, char_b, ocr_w, ocr_b,
                  img_o, rel_o, att_o, name_o, char_o, joint_o, ocr_o):
    def proj(x_ref, w_ref, b_ref):
        return jnp.dot(x_ref[...], w_ref[...],
                       preferred_element_type=jnp.float32) + b_ref[...]

    def l2n(e):
        ss = jnp.sum(e * e, axis=1, keepdims=True)
        return e * jax.lax.rsqrt(jnp.maximum(ss, 1e-24))

    img_e = proj(img_x, img_w, img_b)
    rel_e = proj(rel_x, rel_w, rel_b)
    att_e = proj(att_x, att_w, att_b)
    name_e = proj(name_x, name_w, name_b)
    char_e = proj(char_x, char_w, char_b)
    ocr_e = proj(ocr_x, ocr_w, ocr_b)

    img_o[...] = img_e
    rel_o[...] = rel_e
    att_o[...] = att_e
    name_o[...] = name_e
    char_o[...] = char_e
    ocr_o[...] = ocr_e

    off = 0
    for e, wn_idx in ((img_e, 0), (att_e, 1), (rel_e, 2), (gph_x[...], 3),
                      (name_e, 4), (char_e, 5), (ocr_e, 6)):
        d = e.shape[1]
        joint_o[:, off:off + d] = l2n(e) * wn_ref[wn_idx]
        off += d


def kernel(entity_emb, gc1_w, gc1_b, gc2_w, gc2_b, rel_w, rel_b, att_w, att_b,
           img_w, img_b, name_w, name_b, char_w, char_b, ocr_w, ocr_b,
           fusion_w, input_idx, adj, img_features, rel_features, att_features,
           name_features, char_features, ocr_features):
    n = adj.shape[0]
    n_pad = _round_up(max(n, _ROW_PAD), _ROW_PAD)
    bm = 512 if n_pad % 512 == 0 else 256

    ent_x = _pad_rows(entity_emb[input_idx], n_pad)
    adj_p = _pad_rows(
        jnp.pad(adj, ((0, 0), (0, n_pad - n))) if n != n_pad else adj, n_pad)
    img_x = _pad_rows(img_features, n_pad)
    rel_x = _pad_rows(rel_features, n_pad)
    att_x = _pad_rows(att_features, n_pad)
    name_x = _pad_rows(name_features, n_pad)
    char_x = _pad_rows(char_features, n_pad)
    ocr_x = _pad_rows(ocr_features, n_pad)

    d_in = ent_x.shape[1]
    nhid = gc1_w.shape[1]
    nout = gc2_w.shape[1]
    b1 = gc1_b.reshape(1, -1)
    b2 = gc2_b.reshape(1, -1)

    n_strips = n_pad // bm
    gph = pl.pallas_call(
        functools.partial(_gcn_kernel, bm, n_strips),
        grid=(n_strips + 1,),
        in_specs=[
            pl.BlockSpec((bm, n_pad),
                         lambda i: (jnp.minimum(i, n_strips - 1), 0)),
            pl.BlockSpec((n_pad, d_in), lambda i: (0, 0)),
            pl.BlockSpec((d_in, nhid), lambda i: (0, 0)),
            pl.BlockSpec((1, nhid), lambda i: (0, 0)),
            pl.BlockSpec((nhid, nout), lambda i: (0, 0)),
            pl.BlockSpec((1, nout), lambda i: (0, 0)),
        ],
        out_specs=pl.BlockSpec((n_pad, nout), lambda i: (0, 0)),
        out_shape=jax.ShapeDtypeStruct((n_pad, nout), jnp.float32),
        scratch_shapes=[pltpu.VMEM((n_pad, n_pad), jnp.bfloat16),
                        pltpu.VMEM((n_pad, nout), jnp.bfloat16)],
        compiler_params=pltpu.CompilerParams(
            dimension_semantics=("arbitrary",),
            vmem_limit_bytes=_VMEM_LIMIT),
    )(adj_p, ent_x, gc1_w, b1, gc2_w, b2)

    weight_norm = jax.nn.softmax(fusion_w, axis=0)[:, 0]

    d_img = img_w.shape[1]
    d_rel = rel_w.shape[1]
    d_att = att_w.shape[1]
    d_name = name_w.shape[1]
    d_char = char_w.shape[1]
    d_ocr = ocr_w.shape[1]
    d_joint = d_img + d_att + d_rel + nout + d_name + d_char + d_ocr

    def row_spec(d):
        return pl.BlockSpec((bm, d), lambda i: (i, 0))

    def pinned(shape):
        return pl.BlockSpec(shape, lambda i: (0, 0))

    in_specs = [
        pl.BlockSpec(memory_space=pltpu.MemorySpace.SMEM),
        row_spec(nout),
        row_spec(img_x.shape[1]), row_spec(rel_x.shape[1]),
        row_spec(att_x.shape[1]), row_spec(name_x.shape[1]),
        row_spec(char_x.shape[1]), row_spec(ocr_x.shape[1]),
        pinned(img_w.shape), pinned((1, d_img)),
        pinned(rel_w.shape), pinned((1, d_rel)),
        pinned(att_w.shape), pinned((1, d_att)),
        pinned(name_w.shape), pinned((1, d_name)),
        pinned(char_w.shape), pinned((1, d_char)),
        pinned(ocr_w.shape), pinned((1, d_ocr)),
    ]
    out_specs = (row_spec(d_img), row_spec(d_rel), row_spec(d_att),
                 row_spec(d_name), row_spec(d_char), row_spec(d_joint),
                 row_spec(d_ocr))
    out_shape = tuple(jax.ShapeDtypeStruct((n_pad, d), jnp.float32)
                      for d in (d_img, d_rel, d_att, d_name, d_char,
                                d_joint, d_ocr))

    outs = pl.pallas_call(
        _modal_kernel,
        grid=(n_pad // bm,),
        in_specs=in_specs,
        out_specs=out_specs,
        out_shape=out_shape,
        compiler_params=pltpu.CompilerParams(
            dimension_semantics=("arbitrary",),
            vmem_limit_bytes=_VMEM_LIMIT),
    )(weight_norm, gph,
      img_x, rel_x, att_x, name_x, char_x, ocr_x,
      img_w, img_b.reshape(1, -1), rel_w, rel_b.reshape(1, -1),
      att_w, att_b.reshape(1, -1), name_w, name_b.reshape(1, -1),
      char_w, char_b.reshape(1, -1), ocr_w, ocr_b.reshape(1, -1))

    img_o, rel_o, att_o, name_o, char_o, joint_o, ocr_o = outs
    return (gph[:n], img_o[:n], rel_o[:n], att_o[:n], name_o[:n],
            char_o[:n], joint_o[:n], ocr_o[:n])
